# baseline (speedup 1.0000x reference)
import functools

import jax
import jax.numpy as jnp
from jax.experimental import pallas as pl
from jax.experimental.pallas import tpu as pltpu

N = 4096
H = 16
RB = 256
TW = 32


def _layer_body(h_ref, mask_ref, wsw_ref, wsb_ref, wnw_ref, wnb_ref, a2_ref,
                ow_ref, ob_ref, *refs, first, final):
    if first:
        out_ref, i8_ref, hself_scr, table_scr = refs
    else:
        out_ref, hself_scr, table_scr = refs
        i8_ref = None
    i = pl.program_id(0)

    @pl.when(i == 0)
    def _prologue():
        h = h_ref[...]
        h_self = jnp.dot(h, wsw_ref[...],
                         preferred_element_type=jnp.float32) + wsb_ref[...]
        h_n = jnp.dot(h, wnw_ref[...],
                      preferred_element_type=jnp.float32) + wnb_ref[...]
        s2 = jnp.sum(h_n * a2_ref[...], axis=1, keepdims=True)
        e = jnp.exp(jnp.broadcast_to(s2 - jnp.max(s2), (N, H)))
        hself_scr[...] = h_self
        table_scr[...] = jnp.zeros_like(table_scr)
        table_scr[:, 0:H] = (h_n * e).astype(jnp.bfloat16)
        table_scr[:, H:H + 1] = e[:, 0:1].astype(jnp.bfloat16)

    if first:
        m = mask_ref[...].astype(jnp.bfloat16)
        i8_ref[...] = m
    else:
        m = mask_ref[...]
    acc = jnp.dot(m, table_scr[...],
                  preferred_element_type=jnp.float32)
    num = acc[:, 0:H]
    den = acc[:, H:H + 1]
    den_safe = jnp.where(den > 0.0, den, 1.0)
    h_nb = jnp.where(den > 0.0, num / den_safe, 0.0)
    hs = hself_scr[pl.ds(i * RB, RB), :]
    h_new = jnp.maximum(hs + h_nb, 0.0)
    if final:
        out_ref[...] = jnp.dot(h_new, ow_ref[...],
                               preferred_element_type=jnp.float32) + ob_ref[...]
    else:
        out_ref[...] = h_new


def _layer_call(h, adj, wsw, wsb, wnw, wnb, a2, ow, ob, *, first, final):
    out_cols = ob.shape[1] if final else H
    out_shape = [jax.ShapeDtypeStruct((N, out_cols), jnp.float32)]
    out_specs = [pl.BlockSpec((RB, out_cols), lambda i: (i, 0))]
    if first:
        out_shape.append(jax.ShapeDtypeStruct((N, N), jnp.bfloat16))
        out_specs.append(pl.BlockSpec((RB, N), lambda i: (i, 0)))
    res = pl.pallas_call(
        functools.partial(_layer_body, first=first, final=final),
        grid=(N // RB,),
        in_specs=[
            pl.BlockSpec((N, H), lambda i: (0, 0)),
            pl.BlockSpec((RB, N), lambda i: (i, 0)),
            pl.BlockSpec((H, H), lambda i: (0, 0)),
            pl.BlockSpec((1, H), lambda i: (0, 0)),
            pl.BlockSpec((H, H), lambda i: (0, 0)),
            pl.BlockSpec((1, H), lambda i: (0, 0)),
            pl.BlockSpec((1, H), lambda i: (0, 0)),
            pl.BlockSpec(ow.shape, lambda i: (0, 0)),
            pl.BlockSpec(ob.shape, lambda i: (0, 0)),
        ],
        out_specs=out_specs,
        out_shape=out_shape,
        scratch_shapes=[
            pltpu.VMEM((N, H), jnp.float32),
            pltpu.VMEM((N, TW), jnp.bfloat16),
        ],
        compiler_params=pltpu.CompilerParams(
            dimension_semantics=("arbitrary",),
        ),
    )(h, adj, wsw, wsb, wnw, wnb, a2, ow, ob)
    return res


def kernel(states, adj_matrix, params):
    h = jnp.pad(states, ((0, 0), (0, H - states.shape[1])))
    ow = params['out']['w'].T
    ob = params['out']['b'][None, :]
    mask = adj_matrix
    for l in range(3):
        p = params['l%d' % l]
        wsw = p['ws_w'].T
        wnw = p['wn_w'].T
        if wsw.shape[0] < H:
            pad = ((0, H - wsw.shape[0]), (0, 0))
            wsw = jnp.pad(wsw, pad)
            wnw = jnp.pad(wnw, pad)
        wsb = p['ws_b'][None, :]
        wnb = p['wn_b'][None, :]
        a2 = p['a_w'][0, H:][None, :]
        first, final = l == 0, l == 2
        res = _layer_call(h, mask, wsw, wsb, wnw, wnb, a2, ow, ob,
                          first=first, final=final)
        if first:
            h, mask = res
        else:
            h = res[0]
    return h


# speedup vs baseline: 1.1176x; 1.1176x over previous
import functools

import jax
import jax.numpy as jnp
from jax.experimental import pallas as pl
from jax.experimental.pallas import tpu as pltpu

N = 4096
H = 16
RB = 256
TW = 32


def _layer_body(h_ref, mask_ref, wsw_ref, wsb_ref, wnw_ref, wnb_ref, a2_ref,
                ow_ref, ob_ref, *refs, first, final):
    if first:
        out_ref, i8_ref, hself_scr, table_scr = refs
    else:
        out_ref, hself_scr, table_scr = refs
        i8_ref = None
    i = pl.program_id(0)

    @pl.when(i == 0)
    def _prologue():
        h = h_ref[...]
        h_self = jnp.dot(h, wsw_ref[...],
                         preferred_element_type=jnp.float32) + wsb_ref[...]
        h_n = jnp.dot(h, wnw_ref[...],
                      preferred_element_type=jnp.float32) + wnb_ref[...]
        s2 = jnp.sum(h_n * a2_ref[...], axis=1, keepdims=True)
        e = jnp.exp(jnp.broadcast_to(s2 - jnp.max(s2), (N, H)))
        hself_scr[...] = h_self
        table_scr[...] = jnp.zeros_like(table_scr)
        table_scr[:, 0:H] = (h_n * e).astype(jnp.bfloat16)
        table_scr[:, H:H + 1] = e[:, 0:1].astype(jnp.bfloat16)

    if first:
        m32 = mask_ref[...]
        i8_ref[...] = m32.astype(jnp.int8)
        m = m32.astype(jnp.bfloat16)
    else:
        m = mask_ref[...].astype(jnp.bfloat16)
    acc = jnp.dot(m, table_scr[...],
                  preferred_element_type=jnp.float32)
    num = acc[:, 0:H]
    den = acc[:, H:H + 1]
    den_safe = jnp.where(den > 0.0, den, 1.0)
    h_nb = jnp.where(den > 0.0, num / den_safe, 0.0)
    hs = hself_scr[pl.ds(i * RB, RB), :]
    h_new = jnp.maximum(hs + h_nb, 0.0)
    if final:
        out_ref[...] = jnp.dot(h_new, ow_ref[...],
                               preferred_element_type=jnp.float32) + ob_ref[...]
    else:
        out_ref[...] = h_new


def _layer_call(h, adj, wsw, wsb, wnw, wnb, a2, ow, ob, *, first, final):
    out_cols = ob.shape[1] if final else H
    out_shape = [jax.ShapeDtypeStruct((N, out_cols), jnp.float32)]
    out_specs = [pl.BlockSpec((RB, out_cols), lambda i: (i, 0))]
    if first:
        out_shape.append(jax.ShapeDtypeStruct((N, N), jnp.int8))
        out_specs.append(pl.BlockSpec((RB, N), lambda i: (i, 0)))
    res = pl.pallas_call(
        functools.partial(_layer_body, first=first, final=final),
        grid=(N // RB,),
        in_specs=[
            pl.BlockSpec((N, H), lambda i: (0, 0)),
            pl.BlockSpec((RB, N), lambda i: (i, 0)),
            pl.BlockSpec((H, H), lambda i: (0, 0)),
            pl.BlockSpec((1, H), lambda i: (0, 0)),
            pl.BlockSpec((H, H), lambda i: (0, 0)),
            pl.BlockSpec((1, H), lambda i: (0, 0)),
            pl.BlockSpec((1, H), lambda i: (0, 0)),
            pl.BlockSpec(ow.shape, lambda i: (0, 0)),
            pl.BlockSpec(ob.shape, lambda i: (0, 0)),
        ],
        out_specs=out_specs,
        out_shape=out_shape,
        scratch_shapes=[
            pltpu.VMEM((N, H), jnp.float32),
            pltpu.VMEM((N, TW), jnp.bfloat16),
        ],
        compiler_params=pltpu.CompilerParams(
            dimension_semantics=("arbitrary",),
        ),
    )(h, adj, wsw, wsb, wnw, wnb, a2, ow, ob)
    return res


def kernel(states, adj_matrix, params):
    h = jnp.pad(states, ((0, 0), (0, H - states.shape[1])))
    ow = params['out']['w'].T
    ob = params['out']['b'][None, :]
    mask = adj_matrix
    for l in range(3):
        p = params['l%d' % l]
        wsw = p['ws_w'].T
        wnw = p['wn_w'].T
        if wsw.shape[0] < H:
            pad = ((0, H - wsw.shape[0]), (0, 0))
            wsw = jnp.pad(wsw, pad)
            wnw = jnp.pad(wnw, pad)
        wsb = p['ws_b'][None, :]
        wnb = p['wn_b'][None, :]
        a2 = p['a_w'][0, H:][None, :]
        first, final = l == 0, l == 2
        res = _layer_call(h, mask, wsw, wsb, wnw, wnb, a2, ow, ob,
                          first=first, final=final)
        if first:
            h, mask = res
        else:
            h = res[0]
    return h


# speedup vs baseline: 1.2917x; 1.1558x over previous
import functools

import jax
import jax.numpy as jnp
from jax.experimental import pallas as pl
from jax.experimental.pallas import tpu as pltpu

N = 4096
H = 16
RB = 512
TW = 32


def _layer_body(h_ref, mask_ref, wsw_ref, wsb_ref, wnw_ref, wnb_ref, a2_ref,
                ow_ref, ob_ref, *refs, first, final):
    if first:
        out_ref, i8_ref, hself_scr, table_scr = refs
    else:
        out_ref, hself_scr, table_scr = refs
        i8_ref = None
    i = pl.program_id(0)

    @pl.when(i == 0)
    def _prologue():
        h = h_ref[...]
        h_self = jnp.dot(h, wsw_ref[...],
                         preferred_element_type=jnp.float32) + wsb_ref[...]
        h_n = jnp.dot(h, wnw_ref[...],
                      preferred_element_type=jnp.float32) + wnb_ref[...]
        s2 = jnp.sum(h_n * a2_ref[...], axis=1, keepdims=True)
        e = jnp.exp(jnp.broadcast_to(s2 - jnp.max(s2), (N, H)))
        hself_scr[...] = h_self
        table_scr[...] = jnp.zeros_like(table_scr)
        table_scr[:, 0:H] = (h_n * e).astype(jnp.bfloat16)
        table_scr[:, H:H + 1] = e[:, 0:1].astype(jnp.bfloat16)

    if first:
        m32 = mask_ref[...]
        i8_ref[...] = m32.astype(jnp.int8)
        m = m32.astype(jnp.bfloat16)
    else:
        m = mask_ref[...].astype(jnp.bfloat16)
    acc = jnp.dot(m, table_scr[...],
                  preferred_element_type=jnp.float32)
    num = acc[:, 0:H]
    den = acc[:, H:H + 1]
    den_safe = jnp.where(den > 0.0, den, 1.0)
    h_nb = jnp.where(den > 0.0, num / den_safe, 0.0)
    hs = hself_scr[pl.ds(i * RB, RB), :]
    h_new = jnp.maximum(hs + h_nb, 0.0)
    if final:
        out_ref[...] = jnp.dot(h_new, ow_ref[...],
                               preferred_element_type=jnp.float32) + ob_ref[...]
    else:
        out_ref[...] = h_new


def _layer_call(h, adj, wsw, wsb, wnw, wnb, a2, ow, ob, *, first, final):
    out_cols = ob.shape[1] if final else H
    out_shape = [jax.ShapeDtypeStruct((N, out_cols), jnp.float32)]
    out_specs = [pl.BlockSpec((RB, out_cols), lambda i: (i, 0))]
    if first:
        out_shape.append(jax.ShapeDtypeStruct((N, N), jnp.int8))
        out_specs.append(pl.BlockSpec((RB, N), lambda i: (i, 0)))
    res = pl.pallas_call(
        functools.partial(_layer_body, first=first, final=final),
        grid=(N // RB,),
        in_specs=[
            pl.BlockSpec((N, H), lambda i: (0, 0)),
            pl.BlockSpec((RB, N), lambda i: (i, 0)),
            pl.BlockSpec((H, H), lambda i: (0, 0)),
            pl.BlockSpec((1, H), lambda i: (0, 0)),
            pl.BlockSpec((H, H), lambda i: (0, 0)),
            pl.BlockSpec((1, H), lambda i: (0, 0)),
            pl.BlockSpec((1, H), lambda i: (0, 0)),
            pl.BlockSpec(ow.shape, lambda i: (0, 0)),
            pl.BlockSpec(ob.shape, lambda i: (0, 0)),
        ],
        out_specs=out_specs,
        out_shape=out_shape,
        scratch_shapes=[
            pltpu.VMEM((N, H), jnp.float32),
            pltpu.VMEM((N, TW), jnp.bfloat16),
        ],
        compiler_params=pltpu.CompilerParams(
            dimension_semantics=("arbitrary",),
        ),
    )(h, adj, wsw, wsb, wnw, wnb, a2, ow, ob)
    return res


def kernel(states, adj_matrix, params):
    h = jnp.pad(states, ((0, 0), (0, H - states.shape[1])))
    ow = params['out']['w'].T
    ob = params['out']['b'][None, :]
    mask = adj_matrix
    for l in range(3):
        p = params['l%d' % l]
        wsw = p['ws_w'].T
        wnw = p['wn_w'].T
        if wsw.shape[0] < H:
            pad = ((0, H - wsw.shape[0]), (0, 0))
            wsw = jnp.pad(wsw, pad)
            wnw = jnp.pad(wnw, pad)
        wsb = p['ws_b'][None, :]
        wnb = p['wn_b'][None, :]
        a2 = p['a_w'][0, H:][None, :]
        first, final = l == 0, l == 2
        res = _layer_call(h, mask, wsw, wsb, wnw, wnb, a2, ow, ob,
                          first=first, final=final)
        if first:
            h, mask = res
        else:
            h = res[0]
    return h


# speedup vs baseline: 1.3442x; 1.0407x over previous
import functools

import jax
import jax.numpy as jnp
from jax.experimental import pallas as pl
from jax.experimental.pallas import tpu as pltpu

N = 4096
H = 16
RB = 1024
TW = 32


def _layer_body(h_ref, mask_ref, wsw_ref, wsb_ref, wnw_ref, wnb_ref, a2_ref,
                ow_ref, ob_ref, *refs, first, final):
    if first:
        out_ref, i8_ref, hself_scr, table_scr = refs
    else:
        out_ref, hself_scr, table_scr = refs
        i8_ref = None
    i = pl.program_id(0)

    @pl.when(i == 0)
    def _prologue():
        h = h_ref[...]
        h_self = jnp.dot(h, wsw_ref[...],
                         preferred_element_type=jnp.float32) + wsb_ref[...]
        h_n = jnp.dot(h, wnw_ref[...],
                      preferred_element_type=jnp.float32) + wnb_ref[...]
        s2 = jnp.sum(h_n * a2_ref[...], axis=1, keepdims=True)
        e = jnp.exp(jnp.broadcast_to(s2 - jnp.max(s2), (N, H)))
        hself_scr[...] = h_self
        table_scr[...] = jnp.zeros_like(table_scr)
        table_scr[:, 0:H] = (h_n * e).astype(jnp.bfloat16)
        table_scr[:, H:H + 1] = e[:, 0:1].astype(jnp.bfloat16)

    if first:
        m32 = mask_ref[...]
        i8_ref[...] = m32.astype(jnp.int8)
        m = m32.astype(jnp.bfloat16)
    else:
        m = mask_ref[...].astype(jnp.bfloat16)
    acc = jnp.dot(m, table_scr[...],
                  preferred_element_type=jnp.float32)
    num = acc[:, 0:H]
    den = acc[:, H:H + 1]
    den_safe = jnp.where(den > 0.0, den, 1.0)
    h_nb = jnp.where(den > 0.0, num / den_safe, 0.0)
    hs = hself_scr[pl.ds(i * RB, RB), :]
    h_new = jnp.maximum(hs + h_nb, 0.0)
    if final:
        out_ref[...] = jnp.dot(h_new, ow_ref[...],
                               preferred_element_type=jnp.float32) + ob_ref[...]
    else:
        out_ref[...] = h_new


def _layer_call(h, adj, wsw, wsb, wnw, wnb, a2, ow, ob, *, first, final):
    out_cols = ob.shape[1] if final else H
    out_shape = [jax.ShapeDtypeStruct((N, out_cols), jnp.float32)]
    out_specs = [pl.BlockSpec((RB, out_cols), lambda i: (i, 0))]
    if first:
        out_shape.append(jax.ShapeDtypeStruct((N, N), jnp.int8))
        out_specs.append(pl.BlockSpec((RB, N), lambda i: (i, 0)))
    res = pl.pallas_call(
        functools.partial(_layer_body, first=first, final=final),
        grid=(N // RB,),
        in_specs=[
            pl.BlockSpec((N, H), lambda i: (0, 0)),
            pl.BlockSpec((RB, N), lambda i: (i, 0)),
            pl.BlockSpec((H, H), lambda i: (0, 0)),
            pl.BlockSpec((1, H), lambda i: (0, 0)),
            pl.BlockSpec((H, H), lambda i: (0, 0)),
            pl.BlockSpec((1, H), lambda i: (0, 0)),
            pl.BlockSpec((1, H), lambda i: (0, 0)),
            pl.BlockSpec(ow.shape, lambda i: (0, 0)),
            pl.BlockSpec(ob.shape, lambda i: (0, 0)),
        ],
        out_specs=out_specs,
        out_shape=out_shape,
        scratch_shapes=[
            pltpu.VMEM((N, H), jnp.float32),
            pltpu.VMEM((N, TW), jnp.bfloat16),
        ],
        compiler_params=pltpu.CompilerParams(
            dimension_semantics=("arbitrary",),
        ),
    )(h, adj, wsw, wsb, wnw, wnb, a2, ow, ob)
    return res


def kernel(states, adj_matrix, params):
    h = jnp.pad(states, ((0, 0), (0, H - states.shape[1])))
    ow = params['out']['w'].T
    ob = params['out']['b'][None, :]
    mask = adj_matrix
    for l in range(3):
        p = params['l%d' % l]
        wsw = p['ws_w'].T
        wnw = p['wn_w'].T
        if wsw.shape[0] < H:
            pad = ((0, H - wsw.shape[0]), (0, 0))
            wsw = jnp.pad(wsw, pad)
            wnw = jnp.pad(wnw, pad)
        wsb = p['ws_b'][None, :]
        wnb = p['wn_b'][None, :]
        a2 = p['a_w'][0, H:][None, :]
        first, final = l == 0, l == 2
        res = _layer_call(h, mask, wsw, wsb, wnw, wnb, a2, ow, ob,
                          first=first, final=final)
        if first:
            h, mask = res
        else:
            h = res[0]
    return h
